# baseline (speedup 1.0000x reference)
import jax
import jax.numpy as jnp
from jax import lax
from jax.experimental import pallas as pl
from jax.experimental.pallas import tpu as pltpu
from jax.experimental.pallas import tpu_sc as plsc

B, L, N, D = 4096, 50, 4, 64
NC, NS = 2, 16
NW = NC * NS
BB = 128
LANES = 16
NBG = BB // LANES
ROWS = N * BB


def _body(table_hbm, ids_hbm, out_hbm, idx_v, rows_v, out_v,
          sem0, sem1, osem0, osem1):
    c = lax.axis_index("s") * NC + lax.axis_index("c")
    sems = (sem0, sem1)
    osems = (osem0, osem1)

    def fire(l, slot):
        pltpu.sync_copy(ids_hbm.at[l, c], idx_v.at[slot])
        for j in range(N):
            pltpu.async_copy(table_hbm.at[idx_v.at[slot, j]],
                             rows_v.at[slot, j], sems[slot])

    def drain(slot):
        for j in range(N):
            pltpu.make_async_copy(table_hbm.at[idx_v.at[slot, j]],
                                  rows_v.at[slot, j], sems[slot]).wait()

    def process(l, slot):
        @pl.when(l >= 2)
        def _():
            pltpu.make_async_copy(
                out_v.at[slot],
                out_hbm.at[l - 2, pl.ds(0, D // 8), pl.ds(c, 1)],
                osems[slot]).wait()

        drain(slot)

        def bg_loop(bg, carry):
            bsl = pl.ds(bg * LANES, LANES)
            cnt = jnp.zeros((LANES,), jnp.int32)
            for j in range(N):
                cnt = cnt + jnp.where(idx_v[slot, j, bsl] != 0, 1, 0)
            inv = jnp.where(
                cnt == 0, 0.0,
                jnp.where(cnt == 1, 1.0,
                          jnp.where(cnt == 2, 0.5,
                                    jnp.where(cnt == 3, 1.0 / 3.0, 0.25))))
            inv = inv.astype(jnp.float32)
            bvec = lax.iota(jnp.int32, LANES) + bg * LANES
            for d in range(D):
                dvec = jnp.broadcast_to(jnp.int32(d), (LANES,))
                acc = None
                for j in range(N):
                    v = plsc.load_gather(rows_v.at[slot, j], [bvec, dvec])
                    acc = v if acc is None else acc + v
                out_v[slot, d // 8, 0, d % 8, bsl] = acc * inv
            return carry

        lax.fori_loop(0, NBG, bg_loop, 0)
        pltpu.async_copy(out_v.at[slot],
                         out_hbm.at[l, pl.ds(0, D // 8), pl.ds(c, 1)],
                         osems[slot])

    fire(0, 0)

    def chunk_pair(it, carry):
        for sub in range(2):
            l = 2 * it + sub

            @pl.when(l + 1 < L)
            def _():
                fire(l + 1, 1 - sub)

            process(l, sub)
        return carry

    lax.fori_loop(0, L // 2, chunk_pair, 0)

    for slot in range(2):
        l = L - 2 + slot
        pltpu.make_async_copy(out_v.at[slot],
                              out_hbm.at[l, pl.ds(0, D // 8), pl.ds(c, 1)],
                              osems[slot]).wait()


@jax.jit
def kernel(token_ids, table):
    v = token_ids.transpose((1, 2, 0)).reshape(L, N, NW, BB)
    v = v.transpose((0, 2, 1, 3))

    mesh = plsc.VectorSubcoreMesh(core_axis_name="c", subcore_axis_name="s")
    o5 = pl.kernel(
        _body,
        out_type=jax.ShapeDtypeStruct((L, D // 8, NW, 8, BB), jnp.float32),
        mesh=mesh,
        compiler_params=pltpu.CompilerParams(use_tc_tiling_on_sc=False,
                                             needs_layout_passes=False),
        scratch_types=[
            pltpu.VMEM((2, N, BB), jnp.int32),
            pltpu.VMEM((2, N, BB, D), jnp.float32),
            pltpu.VMEM((2, D // 8, 1, 8, BB), jnp.float32),
            pltpu.SemaphoreType.DMA,
            pltpu.SemaphoreType.DMA,
            pltpu.SemaphoreType.DMA,
            pltpu.SemaphoreType.DMA,
        ],
    )(table, v)
    return o5.transpose((2, 4, 0, 1, 3)).reshape(B, L, D)


# speedup vs baseline: 1.7550x; 1.7550x over previous
import jax
import jax.numpy as jnp
from jax import lax
from jax.experimental import pallas as pl
from jax.experimental.pallas import tpu as pltpu
from jax.experimental.pallas import tpu_sc as plsc

B, L, N, D = 4096, 50, 4, 64
NC, NS = 2, 16
NW = NC * NS
BB = 128
LANES = 16
NBG = BB // LANES
ROWS = N * BB


def _body(table_hbm, ids_hbm, out_hbm, idx_v, rows_v, sums_v, out_v,
          sem0, sem1, osem0, osem1):
    c = lax.axis_index("s") * NC + lax.axis_index("c")
    sems = (sem0, sem1)
    osems = (osem0, osem1)

    def fire(l, slot):
        pltpu.sync_copy(ids_hbm.at[l, c], idx_v.at[slot])
        for j in range(N):
            pltpu.async_copy(table_hbm.at[idx_v.at[slot, j]],
                             rows_v.at[slot, j], sems[slot])

    def drain(slot):
        for j in range(N):
            pltpu.make_async_copy(table_hbm.at[idx_v.at[slot, j]],
                                  rows_v.at[slot, j], sems[slot]).wait()

    def process(l, slot):
        @pl.when(l >= 2)
        def _():
            pltpu.make_async_copy(
                out_v.at[slot],
                out_hbm.at[l - 2, pl.ds(0, D // 8), pl.ds(c, 1)],
                osems[slot]).wait()

        drain(slot)
        iota = lax.iota(jnp.int32, LANES)

        def sum_loop(b, carry):
            for k in range(D // LANES):
                dsl = pl.ds(k * LANES, LANES)
                acc = (rows_v[slot, 0, b, dsl] + rows_v[slot, 1, b, dsl]
                       + rows_v[slot, 2, b, dsl] + rows_v[slot, 3, b, dsl])
                col = (iota + (k * LANES + b)) & (D - 1)
                plsc.store_scatter(sums_v.at[slot], [b * D + col], acc)
            return carry

        lax.fori_loop(0, BB, sum_loop, 0)

        def bg_loop(bg, carry):
            bsl = pl.ds(bg * LANES, LANES)
            cnt = jnp.zeros((LANES,), jnp.int32)
            for j in range(N):
                cnt = cnt + jnp.where(idx_v[slot, j, bsl] != 0, 1, 0)
            inv = jnp.where(
                cnt == 0, 0.0,
                jnp.where(cnt == 1, 1.0,
                          jnp.where(cnt == 2, 0.5,
                                    jnp.where(cnt == 3, 1.0 / 3.0, 0.25))))
            inv = inv.astype(jnp.float32)
            bvec = iota + bg * LANES
            bvec_d = bvec * D
            for d in range(D):
                col = (bvec + d) & (D - 1)
                v = plsc.load_gather(sums_v.at[slot], [bvec_d + col])
                out_v[slot, d // 8, 0, d % 8, bsl] = v * inv
            return carry

        lax.fori_loop(0, NBG, bg_loop, 0)
        pltpu.async_copy(out_v.at[slot],
                         out_hbm.at[l, pl.ds(0, D // 8), pl.ds(c, 1)],
                         osems[slot])

    fire(0, 0)

    def chunk_pair(it, carry):
        for sub in range(2):
            l = 2 * it + sub

            @pl.when(l + 1 < L)
            def _():
                fire(l + 1, 1 - sub)

            process(l, sub)
        return carry

    lax.fori_loop(0, L // 2, chunk_pair, 0)

    for slot in range(2):
        l = L - 2 + slot
        pltpu.make_async_copy(out_v.at[slot],
                              out_hbm.at[l, pl.ds(0, D // 8), pl.ds(c, 1)],
                              osems[slot]).wait()


@jax.jit
def kernel(token_ids, table):
    v = token_ids.transpose((1, 2, 0)).reshape(L, N, NW, BB)
    v = v.transpose((0, 2, 1, 3))

    mesh = plsc.VectorSubcoreMesh(core_axis_name="c", subcore_axis_name="s")
    o5 = pl.kernel(
        _body,
        out_type=jax.ShapeDtypeStruct((L, D // 8, NW, 8, BB), jnp.float32),
        mesh=mesh,
        compiler_params=pltpu.CompilerParams(use_tc_tiling_on_sc=False,
                                             needs_layout_passes=False),
        scratch_types=[
            pltpu.VMEM((2, N, BB), jnp.int32),
            pltpu.VMEM((2, N, BB, D), jnp.float32),
            pltpu.VMEM((2, BB * D), jnp.float32),
            pltpu.VMEM((2, D // 8, 1, 8, BB), jnp.float32),
            pltpu.SemaphoreType.DMA,
            pltpu.SemaphoreType.DMA,
            pltpu.SemaphoreType.DMA,
            pltpu.SemaphoreType.DMA,
        ],
    )(table, v)
    return o5.transpose((2, 4, 0, 1, 3)).reshape(B, L, D)


# speedup vs baseline: 1.8051x; 1.0285x over previous
import jax
import jax.numpy as jnp
from jax import lax
from jax.experimental import pallas as pl
from jax.experimental.pallas import tpu as pltpu
from jax.experimental.pallas import tpu_sc as plsc

B, L, N, D = 4096, 50, 4, 64
NC, NS = 2, 16
NW = NC * NS
BB = 128
LANES = 16
NBG = BB // LANES
ROWS = N * BB


def _body(table_hbm, ids_hbm, out_hbm, idx_v, rows_v, sums_v, out_v,
          sem0, sem1, osem0, osem1):
    c = lax.axis_index("s") * NC + lax.axis_index("c")
    sems = (sem0, sem1)
    osems = (osem0, osem1)

    pltpu.sync_copy(ids_hbm.at[pl.ds(0, L), pl.ds(c, 1)], idx_v)

    def fire(l, slot):
        for j in range(N):
            pltpu.async_copy(table_hbm.at[idx_v.at[l, 0, j]],
                             rows_v.at[slot, j], sems[slot])

    def drain(l, slot):
        for j in range(N):
            pltpu.make_async_copy(table_hbm.at[idx_v.at[l, 0, j]],
                                  rows_v.at[slot, j], sems[slot]).wait()

    def process(l, slot):
        @pl.when(l >= 2)
        def _():
            pltpu.make_async_copy(
                out_v.at[slot],
                out_hbm.at[l - 2, pl.ds(0, D // 8), pl.ds(c, 1)],
                osems[slot]).wait()

        drain(l, slot)
        iota = lax.iota(jnp.int32, LANES)

        def sum_loop(b, carry):
            for k in range(D // LANES):
                dsl = pl.ds(k * LANES, LANES)
                acc = (rows_v[slot, 0, b, dsl] + rows_v[slot, 1, b, dsl]
                       + rows_v[slot, 2, b, dsl] + rows_v[slot, 3, b, dsl])
                col = (iota + (k * LANES + b)) & (D - 1)
                plsc.store_scatter(sums_v.at[slot], [b * D + col], acc)
            return carry

        lax.fori_loop(0, BB, sum_loop, 0)

        def bg_loop(bg, carry):
            bsl = pl.ds(bg * LANES, LANES)
            cnt = jnp.zeros((LANES,), jnp.int32)
            for j in range(N):
                cnt = cnt + jnp.where(idx_v[l, 0, j, bsl] != 0, 1, 0)
            inv = jnp.where(
                cnt == 0, 0.0,
                jnp.where(cnt == 1, 1.0,
                          jnp.where(cnt == 2, 0.5,
                                    jnp.where(cnt == 3, 1.0 / 3.0, 0.25))))
            inv = inv.astype(jnp.float32)
            bvec = iota + bg * LANES
            bvec_d = bvec * D
            for d in range(D):
                col = (bvec + d) & (D - 1)
                v = plsc.load_gather(sums_v.at[slot], [bvec_d + col])
                out_v[slot, d // 8, 0, d % 8, bsl] = v * inv
            return carry

        lax.fori_loop(0, NBG, bg_loop, 0)
        pltpu.async_copy(out_v.at[slot],
                         out_hbm.at[l, pl.ds(0, D // 8), pl.ds(c, 1)],
                         osems[slot])

    fire(0, 0)

    def chunk_pair(it, carry):
        for sub in range(2):
            l = 2 * it + sub

            @pl.when(l + 1 < L)
            def _():
                fire(l + 1, 1 - sub)

            process(l, sub)
        return carry

    lax.fori_loop(0, L // 2, chunk_pair, 0)

    for slot in range(2):
        l = L - 2 + slot
        pltpu.make_async_copy(out_v.at[slot],
                              out_hbm.at[l, pl.ds(0, D // 8), pl.ds(c, 1)],
                              osems[slot]).wait()


@jax.jit
def kernel(token_ids, table):
    v = token_ids.transpose((1, 2, 0)).reshape(L, N, NW, BB)
    v = v.transpose((0, 2, 1, 3))

    mesh = plsc.VectorSubcoreMesh(core_axis_name="c", subcore_axis_name="s")
    o5 = pl.kernel(
        _body,
        out_type=jax.ShapeDtypeStruct((L, D // 8, NW, 8, BB), jnp.float32),
        mesh=mesh,
        compiler_params=pltpu.CompilerParams(use_tc_tiling_on_sc=False,
                                             needs_layout_passes=False),
        scratch_types=[
            pltpu.VMEM((L, 1, N, BB), jnp.int32),
            pltpu.VMEM((2, N, BB, D), jnp.float32),
            pltpu.VMEM((2, BB * D), jnp.float32),
            pltpu.VMEM((2, D // 8, 1, 8, BB), jnp.float32),
            pltpu.SemaphoreType.DMA,
            pltpu.SemaphoreType.DMA,
            pltpu.SemaphoreType.DMA,
            pltpu.SemaphoreType.DMA,
        ],
    )(table, v)
    return o5.transpose((2, 4, 0, 1, 3)).reshape(B, L, D)


# speedup vs baseline: 1.8082x; 1.0017x over previous
import jax
import jax.numpy as jnp
from jax import lax
from jax.experimental import pallas as pl
from jax.experimental.pallas import tpu as pltpu
from jax.experimental.pallas import tpu_sc as plsc

B, L, N, D = 4096, 50, 4, 64
NC, NS = 2, 16
NW = NC * NS
BB = 128
LANES = 16
NBG = BB // LANES
ROWS = N * BB


def _body(table_hbm, ids_hbm, out_hbm, idx_v, rows_v, sums_v, out_v,
          sem0, sem1, osem0, osem1):
    c = lax.axis_index("s") * NC + lax.axis_index("c")
    sems = (sem0, sem1)
    osems = (osem0, osem1)

    pltpu.sync_copy(ids_hbm.at[pl.ds(0, L), pl.ds(c, 1)], idx_v)

    def fire(l, slot):
        for j in range(N):
            pltpu.async_copy(table_hbm.at[idx_v.at[l, 0, j]],
                             rows_v.at[slot, j], sems[slot])

    def drain(l, slot):
        for j in range(N):
            pltpu.make_async_copy(table_hbm.at[idx_v.at[l, 0, j]],
                                  rows_v.at[slot, j], sems[slot]).wait()

    def process(l, slot):
        @pl.when(l >= 2)
        def _():
            pltpu.make_async_copy(
                out_v.at[slot],
                out_hbm.at[l - 2, pl.ds(0, D // 8), pl.ds(c, 1)],
                osems[slot]).wait()

        drain(l, slot)
        iota = lax.iota(jnp.int32, LANES)

        def sum_loop(b, carry):
            for k in range(D // LANES):
                dsl = pl.ds(k * LANES, LANES)
                acc = (rows_v[slot, 0, b, dsl] + rows_v[slot, 1, b, dsl]
                       + rows_v[slot, 2, b, dsl] + rows_v[slot, 3, b, dsl])
                col = (iota + (k * LANES + b)) & (D - 1)
                plsc.store_scatter(sums_v.at[slot], [b * D + col], acc)
            return carry

        lax.fori_loop(0, BB, sum_loop, 0, unroll=4)

        def bg_loop(bg, carry):
            bsl = pl.ds(bg * LANES, LANES)
            cnt = jnp.zeros((LANES,), jnp.int32)
            for j in range(N):
                cnt = cnt + jnp.where(idx_v[l, 0, j, bsl] != 0, 1, 0)
            inv = jnp.where(
                cnt == 0, 0.0,
                jnp.where(cnt == 1, 1.0,
                          jnp.where(cnt == 2, 0.5,
                                    jnp.where(cnt == 3, 1.0 / 3.0, 0.25))))
            inv = inv.astype(jnp.float32)
            bvec = iota + bg * LANES
            bvec_d = bvec * D
            for d in range(D):
                col = (bvec + d) & (D - 1)
                v = plsc.load_gather(sums_v.at[slot], [bvec_d + col])
                out_v[slot, d // 8, 0, d % 8, bsl] = v * inv
            return carry

        lax.fori_loop(0, NBG, bg_loop, 0, unroll=2)
        pltpu.async_copy(out_v.at[slot],
                         out_hbm.at[l, pl.ds(0, D // 8), pl.ds(c, 1)],
                         osems[slot])

    fire(0, 0)

    def chunk_pair(it, carry):
        for sub in range(2):
            l = 2 * it + sub

            @pl.when(l + 1 < L)
            def _():
                fire(l + 1, 1 - sub)

            process(l, sub)
        return carry

    lax.fori_loop(0, L // 2, chunk_pair, 0)

    for slot in range(2):
        l = L - 2 + slot
        pltpu.make_async_copy(out_v.at[slot],
                              out_hbm.at[l, pl.ds(0, D // 8), pl.ds(c, 1)],
                              osems[slot]).wait()


@jax.jit
def kernel(token_ids, table):
    v = token_ids.transpose((1, 2, 0)).reshape(L, N, NW, BB)
    v = v.transpose((0, 2, 1, 3))

    mesh = plsc.VectorSubcoreMesh(core_axis_name="c", subcore_axis_name="s")
    o5 = pl.kernel(
        _body,
        out_type=jax.ShapeDtypeStruct((L, D // 8, NW, 8, BB), jnp.float32),
        mesh=mesh,
        compiler_params=pltpu.CompilerParams(use_tc_tiling_on_sc=False,
                                             needs_layout_passes=False),
        scratch_types=[
            pltpu.VMEM((L, 1, N, BB), jnp.int32),
            pltpu.VMEM((2, N, BB, D), jnp.float32),
            pltpu.VMEM((2, BB * D), jnp.float32),
            pltpu.VMEM((2, D // 8, 1, 8, BB), jnp.float32),
            pltpu.SemaphoreType.DMA,
            pltpu.SemaphoreType.DMA,
            pltpu.SemaphoreType.DMA,
            pltpu.SemaphoreType.DMA,
        ],
    )(table, v)
    return o5.transpose((2, 4, 0, 1, 3)).reshape(B, L, D)
